# Initial kernel scaffold; baseline (speedup 1.0000x reference)
import jax
import jax.numpy as jnp
from jax.experimental import pallas as pl


def kernel(x, edge_index, ln_w, ln_b, W1, b1, W2, b2, Wi, Wh, bi, bh, Wfc, bfc):
    raise NotImplementedError("write your pallas kernel here")


# speedup vs baseline: 1.8502x; 1.8502x over previous
import jax
import jax.numpy as jnp
from jax.experimental import pallas as pl
from jax.experimental.pallas import tpu as pltpu

_INTERPRET = False

_N = 32
_E = 160
_H = 4096
_G = 4 * _H
_RT = 2048
_K = _G // _RT


def _graph_kernel(ei_ref, w1_ref, b1_ref, w2_ref, b2_ref,
                  a2_ref, w12_ref, cmat_ref):
    ei = ei_ref[:]
    loop = jax.lax.broadcasted_iota(jnp.int32, (1, _N), 1)
    srcv = jnp.concatenate([ei[0:1, :], loop], axis=1)
    dstv = jnp.concatenate([ei[1:2, :], loop], axis=1)
    nio = jax.lax.broadcasted_iota(jnp.int32, (_N, _E), 0)
    S = (jnp.broadcast_to(srcv, (_N, _E)) == nio).astype(jnp.float32)
    D = (jnp.broadcast_to(dstv, (_N, _E)) == nio).astype(jnp.float32)
    deg = jnp.sum(D, axis=1, keepdims=True)
    dinv = jax.lax.rsqrt(deg)
    wsrc = jnp.sum(S * dinv, axis=0, keepdims=True)
    wdst = jnp.sum(D * dinv, axis=0, keepdims=True)
    Dw = D * (wsrc * wdst)
    A = jax.lax.dot_general(Dw, S, (((1,), (1,)), ((), ())),
                            preferred_element_type=jnp.float32)
    a2_ref[:] = jnp.dot(A, A, preferred_element_type=jnp.float32)
    w12 = jnp.dot(w1_ref[:], w2_ref[:], preferred_element_type=jnp.float32)
    w12_ref[:] = w12
    arow = jnp.sum(A, axis=1, keepdims=True)
    c1 = jnp.dot(b1_ref[:], w2_ref[:], preferred_element_type=jnp.float32)
    cmat_ref[:] = arow * c1 + b2_ref[:]


def _graph_call(edge_index, W1, b1, W2, b2):
    return pl.pallas_call(
        _graph_kernel,
        out_shape=(
            jax.ShapeDtypeStruct((_N, _N), jnp.float32),
            jax.ShapeDtypeStruct((8, 64), jnp.float32),
            jax.ShapeDtypeStruct((_N, 64), jnp.float32),
        ),
        interpret=_INTERPRET,
    )(edge_index, W1, b1, W2, b2)


def _ln_kernel(x_ref, w_ref, b_ref, w12_ref, z_ref):
    xb = x_ref[:]
    mu = jnp.mean(xb, axis=1, keepdims=True)
    var = jnp.mean((xb - mu) ** 2, axis=1, keepdims=True)
    ln = (xb - mu) * jax.lax.rsqrt(var + 1e-5) * w_ref[:] + b_ref[:]
    z_ref[:] = jnp.dot(ln, w12_ref[:], preferred_element_type=jnp.float32)


def _ln_call(x2d, ln_w, ln_b, W12):
    rows = x2d.shape[0]
    R = 1600
    return pl.pallas_call(
        _ln_kernel,
        grid=(rows // R,),
        in_specs=[
            pl.BlockSpec((R, 8), lambda i: (i, 0)),
            pl.BlockSpec((1, 8), lambda i: (0, 0)),
            pl.BlockSpec((1, 8), lambda i: (0, 0)),
            pl.BlockSpec((8, 64), lambda i: (0, 0)),
        ],
        out_specs=pl.BlockSpec((R, 64), lambda i: (i, 0)),
        out_shape=jax.ShapeDtypeStruct((rows, 64), jnp.float32),
        interpret=_INTERPRET,
    )(x2d, ln_w, ln_b, W12)


def _mix_kernel(a2_ref, z_ref, y_ref):
    y_ref[:] = jnp.dot(a2_ref[:], z_ref[:],
                       preferred_element_type=jnp.float32)


def _mix_call(A2, Z2):
    cols = Z2.shape[1]
    C = 6400
    return pl.pallas_call(
        _mix_kernel,
        grid=(cols // C,),
        in_specs=[
            pl.BlockSpec((_N, _N), lambda i: (0, 0)),
            pl.BlockSpec((_N, C), lambda i: (0, i)),
        ],
        out_specs=pl.BlockSpec((_N, C), lambda i: (0, i)),
        out_shape=jax.ShapeDtypeStruct((_N, cols), jnp.float32),
        interpret=_INTERPRET,
    )(A2, Z2)


def _proj_kernel(y_ref, wi_ref, p_ref):
    yb = y_ref[:].astype(jnp.bfloat16)
    wb = wi_ref[:].astype(jnp.bfloat16)
    p_ref[:] = jax.lax.dot_general(yb, wb, (((1,), (1,)), ((), ())),
                                   preferred_element_type=jnp.float32)


def _proj_call(Ybig, Wi):
    rows = Ybig.shape[0]
    RT = 1024
    return pl.pallas_call(
        _proj_kernel,
        grid=(_G // RT,),
        in_specs=[
            pl.BlockSpec((rows, 2048), lambda k: (0, 0)),
            pl.BlockSpec((RT, 2048), lambda k: (k, 0)),
        ],
        out_specs=pl.BlockSpec((rows, RT), lambda k: (0, k)),
        out_shape=jax.ShapeDtypeStruct((rows, _G), jnp.float32),
        interpret=_INTERPRET,
    )(Ybig, Wi)


def _lstm_kernel(p_ref, pb_ref, bi_ref, bh_ref, wh_ref, wfc_ref, bfc_ref,
                 out_ref, gates, cs, hs, hb):
    t = pl.program_id(0)
    k = pl.program_id(1)
    T = pl.num_programs(0)

    @pl.when(jnp.logical_and(t == 0, k == 0))
    def _init():
        cs[:] = jnp.zeros_like(cs)
        hb[:] = jnp.zeros_like(hb)

    mm = jax.lax.dot_general(hb[:], wh_ref[:], (((1,), (1,)), ((), ())),
                             preferred_element_type=jnp.float32)
    gates[:, pl.ds(k * _RT, _RT)] = mm + p_ref[:] + pb_ref[:] + bi_ref[:] + bh_ref[:]

    @pl.when(k == _K - 1)
    def _update():
        g = gates[:]
        ig = jax.nn.sigmoid(g[:, 0:_H])
        fg = jax.nn.sigmoid(g[:, _H:2 * _H])
        gg = jnp.tanh(g[:, 2 * _H:3 * _H])
        og = jax.nn.sigmoid(g[:, 3 * _H:4 * _H])
        c = fg * cs[:] + ig * gg
        cs[:] = c
        h = og * jnp.tanh(c)
        hs[:] = h
        hb[:] = h.astype(jnp.bfloat16)

    @pl.when(jnp.logical_and(t == T - 1, k == _K - 1))
    def _head():
        hw = hs[:] * wfc_ref[:]
        r = jax.lax.broadcasted_iota(jnp.int32, (_H, _N), 0) // 128
        m = jax.lax.broadcasted_iota(jnp.int32, (_H, _N), 1)
        seg = (r == m).astype(jnp.float32)
        out_ref[:] = jnp.dot(hw, seg,
                             preferred_element_type=jnp.float32) + bfc_ref[:]


def _lstm_call(P, pb, bi, bh, Whb, wfc_t, bfc):
    B = 16
    T = 50
    return pl.pallas_call(
        _lstm_kernel,
        grid=(T, _K),
        in_specs=[
            pl.BlockSpec((B, _RT), lambda t, k: (t, k)),
            pl.BlockSpec((1, _RT), lambda t, k: (0, k)),
            pl.BlockSpec((1, _RT), lambda t, k: (0, k)),
            pl.BlockSpec((1, _RT), lambda t, k: (0, k)),
            pl.BlockSpec((_RT, _H), lambda t, k: (k, 0)),
            pl.BlockSpec((1, _H), lambda t, k: (0, 0)),
            pl.BlockSpec((1, 1), lambda t, k: (0, 0)),
        ],
        out_specs=pl.BlockSpec((B, _N), lambda t, k: (0, 0)),
        out_shape=jax.ShapeDtypeStruct((B, _N), jnp.float32),
        scratch_shapes=[
            pltpu.VMEM((B, _G), jnp.float32),
            pltpu.VMEM((B, _H), jnp.float32),
            pltpu.VMEM((B, _H), jnp.float32),
            pltpu.VMEM((B, _H), jnp.bfloat16),
        ],
        compiler_params=pltpu.CompilerParams(
            dimension_semantics=("arbitrary", "arbitrary")),
        interpret=_INTERPRET,
    )(P, pb, bi, bh, Whb, wfc_t, bfc)


def kernel(x, edge_index, ln_w, ln_b, W1, b1, W2, b2, Wi, Wh, bi, bh, Wfc, bfc):
    B, T, N, F = x.shape

    A2, W12, Cmat = _graph_call(edge_index, W1, b1.reshape(1, 64),
                                W2, b2.reshape(1, 64))

    xT = jnp.transpose(x, (1, 0, 2, 3)).reshape(T * B * N, F)
    Z = _ln_call(xT, ln_w.reshape(1, F), ln_b.reshape(1, F), W12)

    Z2 = Z.reshape(T * B, N, 64).transpose(1, 0, 2).reshape(N, T * B * 64)
    Y2 = _mix_call(A2, Z2)
    Yflat = Y2.reshape(N, T * B, 64).transpose(1, 0, 2).reshape(T * B, N * 64)

    cvec = Cmat.reshape(1, N * 64)
    Ybig = jnp.concatenate(
        [Yflat, cvec, jnp.zeros((7, N * 64), jnp.float32)], axis=0)

    Pbig = _proj_call(Ybig, Wi)
    P, pb = Pbig[:T * B], Pbig[T * B:T * B + 1]

    Whb = Wh.astype(jnp.bfloat16)
    out = _lstm_call(P, pb, bi.reshape(1, _G), bh.reshape(1, _G),
                     Whb, jnp.tile(Wfc, (1, N)), bfc.reshape(1, 1))
    return out.reshape(B, N, 1)
